# baseline (speedup 1.0000x reference)
import functools

import jax
import jax.numpy as jnp
from jax import lax
from jax.experimental import pallas as pl
from jax.experimental.pallas import tpu as pltpu
from jax.experimental.pallas import tpu_sc as plsc

B = 16384
D = 64
DP = D // 2
NEGK = 5
C = 128
LANES = 16


def _pack_bf16_pairs(w):
    lo = jax.lax.bitcast_convert_type(
        w[:, 0::2].astype(jnp.bfloat16), jnp.uint16).astype(jnp.int32)
    hi = jax.lax.bitcast_convert_type(
        w[:, 1::2].astype(jnp.bfloat16), jnp.uint16).astype(jnp.int32)
    return lo | (hi << 16)


def _sc_scores(pos_u, pos_v, neg_flat, wu_p, wv_p):
    info = plsc.get_sparse_core_info()
    nw = info.num_cores * info.num_subcores
    epw = B // nw
    nchunk = epw // C
    mesh = plsc.VectorSubcoreMesh(core_axis_name="c", subcore_axis_name="s")

    @functools.partial(
        pl.kernel,
        out_type=[jax.ShapeDtypeStruct((B,), jnp.float32),
                  jax.ShapeDtypeStruct((B * NEGK,), jnp.float32)],
        mesh=mesh,
        scratch_types=[
            pltpu.VMEM((C,), jnp.int32),
            pltpu.VMEM((C,), jnp.int32),
            pltpu.VMEM((C * NEGK,), jnp.int32),
            pltpu.VMEM((C, DP), jnp.int32),
            pltpu.VMEM((C, DP), jnp.int32),
            pltpu.VMEM((C * NEGK, DP), jnp.int32),
            pltpu.VMEM((C,), jnp.float32),
            pltpu.VMEM((C * NEGK,), jnp.float32),
            pltpu.SemaphoreType.DMA,
        ],
        compiler_params=pltpu.CompilerParams(needs_layout_passes=False,
                                             use_tc_tiling_on_sc=False),
    )
    def scores(pos_u_hbm, pos_v_hbm, neg_hbm, wu_hbm, wv_hbm,
               pos_out, neg_out, iu, iv, ineg, ru, rv, rn, sp, sn, sem):
        wid = lax.axis_index("s") * info.num_cores + lax.axis_index("c")
        lane = jnp.arange(LANES, dtype=jnp.int32)
        mhi = jnp.full((LANES,), -65536, jnp.int32)
        for chunk in range(nchunk):
            b0 = wid * epw + chunk * C
            pltpu.sync_copy(pos_u_hbm.at[pl.ds(b0, C)], iu)
            pltpu.sync_copy(pos_v_hbm.at[pl.ds(b0, C)], iv)
            pltpu.sync_copy(neg_hbm.at[pl.ds(b0 * NEGK, C * NEGK)], ineg)
            cp_u = pltpu.async_copy(wu_hbm.at[iu], ru, sem)
            cp_v = pltpu.async_copy(wv_hbm.at[iv], rv, sem)
            cp_n = pltpu.async_copy(wv_hbm.at[ineg], rn, sem)
            cp_u.wait()
            cp_v.wait()
            cp_n.wait()
            for g in range(C // LANES):
                s = pl.ds(g * LANES, LANES)
                rowu = lane + (g * LANES)
                rown = [rowu * NEGK + n for n in range(NEGK)]

                def dbody(d, accs, rowu=rowu, rown=rown):
                    dcol = jnp.full((LANES,), d, jnp.int32)
                    xu = plsc.load_gather(ru, [rowu, dcol])
                    xv = plsc.load_gather(rv, [rowu, dcol])
                    ul = plsc.bitcast(xu << 16, jnp.float32)
                    uh = plsc.bitcast(xu & mhi, jnp.float32)
                    vl = plsc.bitcast(xv << 16, jnp.float32)
                    vh = plsc.bitcast(xv & mhi, jnp.float32)
                    out = [accs[0] + ul * vl + uh * vh]
                    for n in range(NEGK):
                        xn = plsc.load_gather(rn, [rown[n], dcol])
                        nl = plsc.bitcast(xn << 16, jnp.float32)
                        nh = plsc.bitcast(xn & mhi, jnp.float32)
                        out.append(accs[1 + n] + nl * ul + nh * uh)
                    return tuple(out)

                z = jnp.zeros((LANES,), jnp.float32)
                accs = lax.fori_loop(0, DP, dbody, (z,) * (1 + NEGK))
                sp[s] = accs[0]
                for n in range(NEGK):
                    plsc.store_scatter(sn, [rown[n]], accs[1 + n])
            pltpu.sync_copy(sp, pos_out.at[pl.ds(b0, C)])
            pltpu.sync_copy(sn, neg_out.at[pl.ds(b0 * NEGK, C * NEGK)])

    return scores(pos_u, pos_v, neg_flat, wu_p, wv_p)


def _loss(pos_s, neg_s):
    pos2 = pos_s.reshape(B // 128, 128)
    neg2 = neg_s.reshape(B * NEGK // 128, 128)

    def body(p_ref, n_ref, o_ref):
        p = jnp.clip(p_ref[...], -6.0, 6.0)
        n = jnp.clip(n_ref[...], -6.0, 6.0)
        lp = jnp.log1p(jnp.exp(-p))
        ln = jnp.log1p(jnp.exp(n))
        o_ref[0, 0] = (jnp.sum(lp) + jnp.sum(ln)) * (1.0 / B)

    out = pl.pallas_call(
        body,
        out_shape=jax.ShapeDtypeStruct((1, 1), jnp.float32),
        out_specs=pl.BlockSpec(memory_space=pltpu.SMEM),
    )(pos2, neg2)
    return out[0, 0]


def kernel(pos_u, pos_v, neg_v, snd_u_weight, snd_v_weight):
    wu_p = _pack_bf16_pairs(snd_u_weight)
    wv_p = _pack_bf16_pairs(snd_v_weight)
    pos_s, neg_s = _sc_scores(pos_u, pos_v, neg_v.reshape(-1), wu_p, wv_p)
    return _loss(pos_s, neg_s)


# speedup vs baseline: 3.2222x; 3.2222x over previous
import functools

import jax
import jax.numpy as jnp
from jax import lax
from jax.experimental import pallas as pl
from jax.experimental.pallas import tpu as pltpu
from jax.experimental.pallas import tpu_sc as plsc

B = 16384
D = 64
NEGK = 5
C = 128
LANES = 16
UNROLL = 4


def _sc_scores(pos_u, pos_v, neg_flat, wu, wv):
    info = plsc.get_sparse_core_info()
    nw = info.num_cores * info.num_subcores
    epw = B // nw
    nchunk = epw // C
    mesh = plsc.VectorSubcoreMesh(core_axis_name="c", subcore_axis_name="s")

    buf = lambda shape, dt: [pltpu.VMEM(shape, dt) for _ in range(2)]

    @functools.partial(
        pl.kernel,
        out_type=[jax.ShapeDtypeStruct((B,), jnp.float32),
                  jax.ShapeDtypeStruct((B * NEGK,), jnp.float32)],
        mesh=mesh,
        scratch_types=[
            buf((C,), jnp.int32),
            buf((C,), jnp.int32),
            buf((C * NEGK,), jnp.int32),
            buf((C, D), jnp.float32),
            buf((C, D), jnp.float32),
            buf((C * NEGK, D), jnp.float32),
            pltpu.VMEM((C,), jnp.float32),
            pltpu.VMEM((C * NEGK,), jnp.float32),
            [pltpu.SemaphoreType.DMA for _ in range(2)],
        ],
        compiler_params=pltpu.CompilerParams(needs_layout_passes=False,
                                             use_tc_tiling_on_sc=False),
    )
    def scores(pos_u_hbm, pos_v_hbm, neg_hbm, wu_hbm, wv_hbm,
               pos_out, neg_out, iu, iv, ineg, ru, rv, rn, sp, sn, sem):
        wid = lax.axis_index("s") * info.num_cores + lax.axis_index("c")
        lane = jnp.arange(LANES, dtype=jnp.int32)

        def stage(c):
            p = c % 2
            b0 = wid * epw + c * C
            pltpu.sync_copy(pos_u_hbm.at[pl.ds(b0, C)], iu[p])
            pltpu.sync_copy(pos_v_hbm.at[pl.ds(b0, C)], iv[p])
            pltpu.sync_copy(neg_hbm.at[pl.ds(b0 * NEGK, C * NEGK)], ineg[p])
            return [pltpu.async_copy(wu_hbm.at[iu[p]], ru[p], sem[p]),
                    pltpu.async_copy(wv_hbm.at[iv[p]], rv[p], sem[p]),
                    pltpu.async_copy(wv_hbm.at[ineg[p]], rn[p], sem[p])]

        cps = stage(0)
        for c in range(nchunk):
            p = c % 2
            for cp in cps:
                cp.wait()
            if c + 1 < nchunk:
                cps = stage(c + 1)
            b0 = wid * epw + c * C
            for g in range(C // LANES):
                s = pl.ds(g * LANES, LANES)
                rowu = lane + (g * LANES)
                rown = [rowu * NEGK + n for n in range(NEGK)]

                def dbody(j, accs, p=p, rowu=rowu, rown=rown):
                    out = list(accs)
                    for k in range(UNROLL):
                        dcol = jnp.full((LANES,), j * UNROLL + k, jnp.int32)
                        xu = plsc.load_gather(ru[p], [rowu, dcol])
                        xv = plsc.load_gather(rv[p], [rowu, dcol])
                        out[0] = out[0] + xu * xv
                        for n in range(NEGK):
                            xn = plsc.load_gather(rn[p], [rown[n], dcol])
                            out[1 + n] = out[1 + n] + xn * xu
                    return tuple(out)

                z = jnp.zeros((LANES,), jnp.float32)
                accs = lax.fori_loop(0, D // UNROLL, dbody,
                                     (z,) * (1 + NEGK))
                sp[s] = accs[0]
                for n in range(NEGK):
                    plsc.store_scatter(sn, [rown[n]], accs[1 + n])
            pltpu.sync_copy(sp, pos_out.at[pl.ds(b0, C)])
            pltpu.sync_copy(sn, neg_out.at[pl.ds(b0 * NEGK, C * NEGK)])

    return scores(pos_u, pos_v, neg_flat, wu, wv)


def _loss(pos_s, neg_s):
    pos2 = pos_s.reshape(B // 128, 128)
    neg2 = neg_s.reshape(B * NEGK // 128, 128)

    def body(p_ref, n_ref, o_ref):
        p = jnp.clip(p_ref[...], -6.0, 6.0)
        n = jnp.clip(n_ref[...], -6.0, 6.0)
        lp = jnp.log1p(jnp.exp(-p))
        ln = jnp.log1p(jnp.exp(n))
        o_ref[0, 0] = (jnp.sum(lp) + jnp.sum(ln)) * (1.0 / B)

    out = pl.pallas_call(
        body,
        out_shape=jax.ShapeDtypeStruct((1, 1), jnp.float32),
        out_specs=pl.BlockSpec(memory_space=pltpu.SMEM),
    )(pos2, neg2)
    return out[0, 0]


def kernel(pos_u, pos_v, neg_v, snd_u_weight, snd_v_weight):
    pos_s, neg_s = _sc_scores(pos_u, pos_v, neg_v.reshape(-1),
                              snd_u_weight, snd_v_weight)
    return _loss(pos_s, neg_s)
